# Initial kernel scaffold; baseline (speedup 1.0000x reference)
import jax
import jax.numpy as jnp
from jax.experimental import pallas as pl


def kernel(x, batch, emb_table, W1, b1, W2, b2, W3, b3, W4, b4):
    raise NotImplementedError("write your pallas kernel here")


# speedup vs baseline: 4.2407x; 4.2407x over previous
import functools

import jax
import jax.numpy as jnp
from jax import lax
from jax.experimental import pallas as pl
from jax.experimental.pallas import tpu as pltpu
from jax.experimental.pallas import tpu_sc as plsc

NUM_SEG = 10000
N_ELEM = 320000
X_DIM = 128
Y_DIM = 64
V_ROWS = 100
V_PAD = 128

NC, NS = 2, 16
NW = NC * NS
CHUNK = 128
NCHUNKS = 80
EPW = NCHUNKS * CHUNK
N_PAD = EPW * NW
ACC_ROWS = 10240
ROWS_PER_TILE = ACC_ROWS // NS
OUT_ROWS_PER_TILE = NUM_SEG // NS


def _table_body(emb_ref, w1_ref, b1_ref, w2_ref, b2_ref, out_ref):
    h = jnp.dot(emb_ref[...], w1_ref[...],
                preferred_element_type=jnp.float32)
    h = jnp.maximum(h + b1_ref[...], 0.0)
    out_ref[...] = jnp.dot(h, w2_ref[...],
                           preferred_element_type=jnp.float32) + b2_ref[...]


def _make_table2(emb_pad, W1, b1, W2, b2):
    return pl.pallas_call(
        _table_body,
        out_shape=jax.ShapeDtypeStruct((V_PAD, Y_DIM), jnp.float32),
    )(emb_pad, W1, b1.reshape(1, X_DIM), W2, b2.reshape(1, Y_DIM))


def _scatter_body(x_hbm, b_hbm, tab_hbm, out_hbm,
                  xi_v, bi_v, rows_v, zer_v, acc_sh, sem):
    cid = lax.axis_index("c")
    sid = lax.axis_index("s")
    wid = sid * NC + cid

    for r in range(16):
        for c in range(Y_DIM // 16):
            zer_v[r, pl.ds(c * 16, 16)] = jnp.zeros((16,), jnp.float32)
    n_rep = ROWS_PER_TILE // 16
    @pl.loop(0, n_rep)
    def _zero(i):
        pltpu.sync_copy(zer_v, acc_sh.at[pl.ds(sid * ROWS_PER_TILE + i * 16, 16)])
    plsc.subcore_barrier()

    pltpu.sync_copy(x_hbm.at[pl.ds(wid * NCHUNKS, NCHUNKS)], xi_v)
    pltpu.sync_copy(b_hbm.at[pl.ds(wid * NCHUNKS, NCHUNKS)], bi_v)

    @pl.loop(0, NCHUNKS)
    def _acc(j):
        pltpu.async_copy(tab_hbm.at[xi_v.at[j]], rows_v, sem).wait()
        pltpu.sync_copy(rows_v, acc_sh.at[bi_v.at[j]], add=True)

    plsc.subcore_barrier()
    pltpu.sync_copy(acc_sh.at[pl.ds(sid * ROWS_PER_TILE, ROWS_PER_TILE)],
                    out_hbm.at[cid, pl.ds(sid * ROWS_PER_TILE, ROWS_PER_TILE)])


@functools.cache
def _scatter_call():
    return pl.kernel(
        _scatter_body,
        out_type=jax.ShapeDtypeStruct((NC, ACC_ROWS, Y_DIM), jnp.float32),
        mesh=plsc.VectorSubcoreMesh(core_axis_name="c", subcore_axis_name="s",
                                    num_cores=NC, num_subcores=NS),
        compiler_params=pltpu.CompilerParams(use_tc_tiling_on_sc=False),
        scratch_types=[
            pltpu.VMEM((NCHUNKS, CHUNK), jnp.int32),
            pltpu.VMEM((NCHUNKS, CHUNK), jnp.int32),
            pltpu.VMEM((CHUNK, Y_DIM), jnp.float32),
            pltpu.VMEM((16, Y_DIM), jnp.float32),
            pltpu.VMEM_SHARED((ACC_ROWS, Y_DIM), jnp.float32),
            pltpu.SemaphoreType.DMA,
        ],
    )


def _head_body(acc_ref, w3_ref, b3_ref, w4_ref, b4_ref, out_ref):
    s = acc_ref[0, :NUM_SEG] + acc_ref[1, :NUM_SEG]
    h = jnp.dot(s, w3_ref[...], preferred_element_type=jnp.float32)
    h = jnp.maximum(h + b3_ref[...], 0.0)
    out_ref[...] = (
        jnp.sum(h * w4_ref[...], axis=1, keepdims=True) + b4_ref[...]
    )


def _head(acc, W3, b3, W4, b4):
    return pl.pallas_call(
        _head_body,
        out_shape=jax.ShapeDtypeStruct((NUM_SEG, 1), jnp.float32),
    )(acc, W3, b3.reshape(1, Y_DIM), W4.reshape(1, Y_DIM), b4.reshape(1, 1))


def kernel(x, batch, emb_table, W1, b1, W2, b2, W3, b3, W4, b4):
    emb_pad = jnp.pad(emb_table, ((0, V_PAD - V_ROWS), (0, 0)))
    table2 = _make_table2(emb_pad, W1, b1, W2, b2)

    xp = jnp.pad(x.astype(jnp.int32), (0, N_PAD - N_ELEM))
    bp = jnp.pad(batch.astype(jnp.int32), (0, N_PAD - N_ELEM),
                 constant_values=NUM_SEG)
    x2 = xp.reshape(NW * NCHUNKS, CHUNK)
    b2d = bp.reshape(NW * NCHUNKS, CHUNK)

    acc = _scatter_call()(x2, b2d, table2)
    return _head(acc, W3, b3, W4, b4)


# speedup vs baseline: 4.5601x; 1.0753x over previous
import functools

import jax
import jax.numpy as jnp
from jax import lax
from jax.experimental import pallas as pl
from jax.experimental.pallas import tpu as pltpu
from jax.experimental.pallas import tpu_sc as plsc

NUM_SEG = 10000
N_ELEM = 320000
X_DIM = 128
Y_DIM = 64
V_ROWS = 100
V_PAD = 128

NC, NS = 2, 16
NW = NC * NS
CHUNK = 128
NCHUNKS = 80
EPW = NCHUNKS * CHUNK
N_PAD = EPW * NW
ACC_ROWS = 10240
ROWS_PER_TILE = ACC_ROWS // NS
OUT_ROWS_PER_TILE = NUM_SEG // NS


def _table_body(emb_ref, w1_ref, b1_ref, w2_ref, b2_ref, out_ref):
    h = jnp.dot(emb_ref[...], w1_ref[...],
                preferred_element_type=jnp.float32)
    h = jnp.maximum(h + b1_ref[...], 0.0)
    out_ref[...] = jnp.dot(h, w2_ref[...],
                           preferred_element_type=jnp.float32) + b2_ref[...]


def _make_table2(emb_pad, W1, b1, W2, b2):
    return pl.pallas_call(
        _table_body,
        out_shape=jax.ShapeDtypeStruct((V_PAD, Y_DIM), jnp.float32),
    )(emb_pad, W1, b1.reshape(1, X_DIM), W2, b2.reshape(1, Y_DIM))


def _scatter_body(x_hbm, b_hbm, tab_hbm, out_hbm,
                  xi_v, bi_v, rows_v, zer_v, acc_sh, sem0, sem1):
    cid = lax.axis_index("c")
    sid = lax.axis_index("s")
    wid = sid * NC + cid

    for r in range(16):
        for c in range(Y_DIM // 16):
            zer_v[r, pl.ds(c * 16, 16)] = jnp.zeros((16,), jnp.float32)
    n_rep = ROWS_PER_TILE // 16
    @pl.loop(0, n_rep)
    def _zero(i):
        pltpu.sync_copy(zer_v, acc_sh.at[pl.ds(sid * ROWS_PER_TILE + i * 16, 16)])
    plsc.subcore_barrier()

    pltpu.sync_copy(x_hbm.at[pl.ds(wid * NCHUNKS, NCHUNKS)], xi_v)
    pltpu.sync_copy(b_hbm.at[pl.ds(wid * NCHUNKS, NCHUNKS)], bi_v)

    def _gather(j, buf, sem):
        pltpu.async_copy(tab_hbm.at[xi_v.at[j]], rows_v.at[buf], sem)

    def _gwait(buf, sem):
        pltpu.make_async_copy(tab_hbm.at[xi_v.at[0]], rows_v.at[buf], sem).wait()

    def _scatter(j, buf):
        pltpu.sync_copy(rows_v.at[buf], acc_sh.at[bi_v.at[j]], add=True)

    _gather(0, 0, sem0)

    @pl.loop(0, (NCHUNKS - 2) // 2)
    def _acc(k):
        j = 2 * k
        _gather(j + 1, 1, sem1)
        _gwait(0, sem0)
        _scatter(j, 0)
        _gather(j + 2, 0, sem0)
        _gwait(1, sem1)
        _scatter(j + 1, 1)

    _gather(NCHUNKS - 1, 1, sem1)
    _gwait(0, sem0)
    _scatter(NCHUNKS - 2, 0)
    _gwait(1, sem1)
    _scatter(NCHUNKS - 1, 1)

    plsc.subcore_barrier()
    pltpu.sync_copy(acc_sh.at[pl.ds(sid * ROWS_PER_TILE, ROWS_PER_TILE)],
                    out_hbm.at[cid, pl.ds(sid * ROWS_PER_TILE, ROWS_PER_TILE)])


@functools.cache
def _scatter_call():
    return pl.kernel(
        _scatter_body,
        out_type=jax.ShapeDtypeStruct((NC, ACC_ROWS, Y_DIM), jnp.float32),
        mesh=plsc.VectorSubcoreMesh(core_axis_name="c", subcore_axis_name="s",
                                    num_cores=NC, num_subcores=NS),
        compiler_params=pltpu.CompilerParams(use_tc_tiling_on_sc=False),
        scratch_types=[
            pltpu.VMEM((NCHUNKS, CHUNK), jnp.int32),
            pltpu.VMEM((NCHUNKS, CHUNK), jnp.int32),
            pltpu.VMEM((2, CHUNK, Y_DIM), jnp.float32),
            pltpu.VMEM((16, Y_DIM), jnp.float32),
            pltpu.VMEM_SHARED((ACC_ROWS, Y_DIM), jnp.float32),
            pltpu.SemaphoreType.DMA,
            pltpu.SemaphoreType.DMA,
        ],
    )


def _head_body(acc_ref, w3_ref, b3_ref, w4_ref, b4_ref, out_ref):
    s = acc_ref[0, :NUM_SEG] + acc_ref[1, :NUM_SEG]
    h = jnp.dot(s, w3_ref[...], preferred_element_type=jnp.float32)
    h = jnp.maximum(h + b3_ref[...], 0.0)
    out_ref[...] = (
        jnp.sum(h * w4_ref[...], axis=1, keepdims=True) + b4_ref[...]
    )


def _head(acc, W3, b3, W4, b4):
    return pl.pallas_call(
        _head_body,
        out_shape=jax.ShapeDtypeStruct((NUM_SEG, 1), jnp.float32),
    )(acc, W3, b3.reshape(1, Y_DIM), W4.reshape(1, Y_DIM), b4.reshape(1, 1))


def kernel(x, batch, emb_table, W1, b1, W2, b2, W3, b3, W4, b4):
    emb_pad = jnp.pad(emb_table, ((0, V_PAD - V_ROWS), (0, 0)))
    table2 = _make_table2(emb_pad, W1, b1, W2, b2)

    xp = jnp.pad(x.astype(jnp.int32), (0, N_PAD - N_ELEM))
    bp = jnp.pad(batch.astype(jnp.int32), (0, N_PAD - N_ELEM),
                 constant_values=NUM_SEG)
    x2 = xp.reshape(NW * NCHUNKS, CHUNK)
    b2d = bp.reshape(NW * NCHUNKS, CHUNK)

    acc = _scatter_call()(x2, b2d, table2)
    return _head(acc, W3, b3, W4, b4)


# speedup vs baseline: 11.8780x; 2.6048x over previous
import functools

import jax
import jax.numpy as jnp
from jax import lax
from jax.experimental import pallas as pl
from jax.experimental.pallas import tpu as pltpu
from jax.experimental.pallas import tpu_sc as plsc

NUM_SEG = 10000
N_ELEM = 320000
X_DIM = 128
Y_DIM = 64
V_ROWS = 100
V_PAD = 128

NC, NS = 2, 16
NW = NC * NS
BLK = 128
NBLK = 80
EPW = NBLK * BLK
N_PAD = EPW * NW
ACC_ROWS = 10240
ROWS_PER_TILE = ACC_ROWS // NS
DUMMY = ACC_ROWS - 1
FCAP = 256


def _table_body(emb_ref, w1_ref, b1_ref, w2_ref, b2_ref, out_ref):
    h = jnp.dot(emb_ref[...], w1_ref[...],
                preferred_element_type=jnp.float32)
    h = jnp.maximum(h + b1_ref[...], 0.0)
    out_ref[...] = jnp.dot(h, w2_ref[...],
                           preferred_element_type=jnp.float32) + b2_ref[...]


def _make_table2(emb_pad, W1, b1, W2, b2):
    return pl.pallas_call(
        _table_body,
        out_shape=jax.ShapeDtypeStruct((V_PAD, Y_DIM), jnp.float32),
    )(emb_pad, W1, b1.reshape(1, X_DIM), W2, b2.reshape(1, Y_DIM))


def _seg_body(x_hbm, b_hbm, tab_hbm, out_hbm,
              xi_v, bi_v, tab_v, frows_v, fidx_v, zer_v, acc_sh):
    cid = lax.axis_index("c")
    sid = lax.axis_index("s")
    wid = sid * NC + cid

    for r in range(16):
        for c in range(Y_DIM // 16):
            zer_v[r, pl.ds(c * 16, 16)] = jnp.zeros((16,), jnp.float32)

    @pl.loop(0, ROWS_PER_TILE // 16)
    def _zero(i):
        pltpu.sync_copy(zer_v, acc_sh.at[pl.ds(sid * ROWS_PER_TILE + i * 16, 16)])
    plsc.subcore_barrier()

    pltpu.sync_copy(tab_hbm, tab_v)
    pltpu.sync_copy(x_hbm.at[pl.ds(wid * EPW, EPW)], xi_v)
    pltpu.sync_copy(b_hbm.at[pl.ds(wid * EPW, EPW)], bi_v)

    zvec = jnp.zeros((16,), jnp.float32)
    lane = lax.iota(jnp.int32, 16)

    def _drain(d, n, gran):
        @pl.loop(0, (n - d) // gran)
        def _fire(q):
            dd = d + q * gran
            off = pl.multiple_of(dd % FCAP, 16)
            pltpu.sync_copy(
                frows_v.at[pl.ds(off, gran)],
                acc_sh.at[fidx_v.at[pl.ds(off, gran)]],
                add=True)
        return d + ((n - d) // gran) * gran

    def _flush(n, b_prev, fvec, a0, a1, a2, a3):
        s = n % FCAP
        ridx = jnp.full((16,), s, jnp.int32)
        plsc.store_scatter(frows_v, [ridx, lane], a0)
        plsc.store_scatter(frows_v, [ridx, lane + 16], a1)
        plsc.store_scatter(frows_v, [ridx, lane + 32], a2)
        plsc.store_scatter(frows_v, [ridx, lane + 48], a3)
        fvec = jnp.where(lane == s % 16, jnp.full((16,), b_prev, jnp.int32),
                         fvec)
        fidx_v[pl.ds(pl.multiple_of((s // 16) * 16, 16), 16)] = fvec
        return fvec

    def _block(j, carry):
        b_prev, n, d, fvec, a0, a1, a2, a3 = carry

        def _group(g, ecarry):
            b_prev, n, fvec, a0, a1, a2, a3 = ecarry
            i = (j * (BLK // 16) + g) * 16
            xv16 = xi_v[pl.ds(i, 16)]
            bv16 = bi_v[pl.ds(i, 16)]
            for k in range(16):
                xv = xv16[k]
                bv = bv16[k]
                new = bv != b_prev
                fvec = _flush(n, b_prev, fvec, a0, a1, a2, a3)
                n = n + new.astype(jnp.int32)
                base = xv * Y_DIM
                t0 = tab_v[pl.ds(base, 16)]
                t1 = tab_v[pl.ds(base + 16, 16)]
                t2 = tab_v[pl.ds(base + 32, 16)]
                t3 = tab_v[pl.ds(base + 48, 16)]
                a0 = t0 + jnp.where(new, zvec, a0)
                a1 = t1 + jnp.where(new, zvec, a1)
                a2 = t2 + jnp.where(new, zvec, a2)
                a3 = t3 + jnp.where(new, zvec, a3)
                b_prev = bv
            return b_prev, n, fvec, a0, a1, a2, a3

        b_prev, n, fvec, a0, a1, a2, a3 = lax.fori_loop(
            0, BLK // 16, _group, (b_prev, n, fvec, a0, a1, a2, a3))
        d = _drain(d, n, 64)
        return b_prev, n, d, fvec, a0, a1, a2, a3

    izvec = jnp.zeros((16,), jnp.int32)
    init = (jnp.int32(DUMMY), jnp.int32(0), jnp.int32(0), izvec,
            zvec, zvec, zvec, zvec)
    carry = lax.fori_loop(0, NBLK, _block, init)
    b_prev, n, d, fvec, a0, a1, a2, a3 = carry

    _flush(n, b_prev, fvec, a0, a1, a2, a3)
    n = n + 1
    d = _drain(d, n, 16)
    rem = n - d

    @pl.when(rem > 0)
    def _tail():
        g = pl.multiple_of(d % FCAP, 16)
        row = fidx_v[pl.ds(g, 16)]
        keep = lane < rem
        fidx_v[pl.ds(g, 16)] = jnp.where(keep, row,
                                         jnp.full((16,), DUMMY, jnp.int32))
        pltpu.sync_copy(frows_v.at[pl.ds(g, 16)],
                        acc_sh.at[fidx_v.at[pl.ds(g, 16)]], add=True)

    plsc.subcore_barrier()
    pltpu.sync_copy(acc_sh.at[pl.ds(sid * ROWS_PER_TILE, ROWS_PER_TILE)],
                    out_hbm.at[cid, pl.ds(sid * ROWS_PER_TILE, ROWS_PER_TILE)])


@functools.cache
def _seg_call():
    return pl.kernel(
        _seg_body,
        out_type=jax.ShapeDtypeStruct((NC, ACC_ROWS, Y_DIM), jnp.float32),
        mesh=plsc.VectorSubcoreMesh(core_axis_name="c", subcore_axis_name="s",
                                    num_cores=NC, num_subcores=NS),
        compiler_params=pltpu.CompilerParams(use_tc_tiling_on_sc=False,
                                             needs_layout_passes=False),
        scratch_types=[
            pltpu.VMEM((EPW,), jnp.int32),
            pltpu.VMEM((EPW,), jnp.int32),
            pltpu.VMEM((V_PAD * Y_DIM,), jnp.float32),
            pltpu.VMEM((FCAP, Y_DIM), jnp.float32),
            pltpu.VMEM((FCAP,), jnp.int32),
            pltpu.VMEM((16, Y_DIM), jnp.float32),
            pltpu.VMEM_SHARED((ACC_ROWS, Y_DIM), jnp.float32),
        ],
    )


def _head_body(acc_ref, w3_ref, b3_ref, w4_ref, b4_ref, out_ref):
    s = acc_ref[0, :NUM_SEG] + acc_ref[1, :NUM_SEG]
    h = jnp.dot(s, w3_ref[...], preferred_element_type=jnp.float32)
    h = jnp.maximum(h + b3_ref[...], 0.0)
    out_ref[...] = (
        jnp.sum(h * w4_ref[...], axis=1, keepdims=True) + b4_ref[...]
    )


def _head(acc, W3, b3, W4, b4):
    return pl.pallas_call(
        _head_body,
        out_shape=jax.ShapeDtypeStruct((NUM_SEG, 1), jnp.float32),
    )(acc, W3, b3.reshape(1, Y_DIM), W4.reshape(1, Y_DIM), b4.reshape(1, 1))


def kernel(x, batch, emb_table, W1, b1, W2, b2, W3, b3, W4, b4):
    emb_pad = jnp.pad(emb_table, ((0, V_PAD - V_ROWS), (0, 0)))
    table2 = _make_table2(emb_pad, W1, b1, W2, b2)

    xp = jnp.pad(x.astype(jnp.int32), (0, N_PAD - N_ELEM))
    bp = jnp.pad(batch.astype(jnp.int32), (0, N_PAD - N_ELEM),
                 constant_values=NUM_SEG)

    acc = _seg_call()(xp, bp, table2.reshape(-1))
    return _head(acc, W3, b3, W4, b4)
